# Initial kernel scaffold; baseline (speedup 1.0000x reference)
import jax
import jax.numpy as jnp
from jax.experimental import pallas as pl


def kernel(x, edge_index, batch, W1, b1, W2, b2, W3, b3, Wl, bl):
    raise NotImplementedError("write your pallas kernel here")


# speedup vs baseline: 6.7306x; 6.7306x over previous
import functools

import jax
import jax.numpy as jnp
from jax import lax
from jax.experimental import pallas as pl
from jax.experimental.pallas import tpu as pltpu
from jax.experimental.pallas import tpu_sc as plsc

N = 10000
E = 320000
D = 128
H = 128
G = 64

NC = 2
NS = 16
NW = NC * NS
B = 128
NB = 80
EP = NW * NB * B
NP = 10240
RPS = NP // NS
BM = 1024

@functools.lru_cache(maxsize=None)
def _vector_mesh():
    return plsc.VectorSubcoreMesh(
        core_axis_name="c", subcore_axis_name="s", num_cores=NC, num_subcores=NS
    )


def _hist_body(dst_hbm, ones_hbm, zeros_hbm, out_hbm, acc_sh, dst_v, ones_v):
    c = lax.axis_index("c")
    s = lax.axis_index("s")
    wid = c * NS + s
    pltpu.sync_copy(zeros_hbm, acc_sh.at[pl.ds(s * RPS, RPS)])
    pltpu.sync_copy(ones_hbm, ones_v)
    pltpu.sync_copy(dst_hbm.at[wid], dst_v)
    plsc.subcore_barrier()

    @pl.loop(0, NB)
    def _(j):
        pltpu.sync_copy(ones_v, acc_sh.at[dst_v.at[j]], add=True)

    plsc.subcore_barrier()
    pltpu.sync_copy(
        acc_sh.at[pl.ds(s * RPS, RPS)], out_hbm.at[c, pl.ds(s * RPS, RPS)]
    )


@functools.lru_cache(maxsize=None)
def _hist_kernel_fn():
    return pl.kernel(
        _hist_body,
        out_type=jax.ShapeDtypeStruct((NC, NP, H), jnp.float32),
        mesh=_vector_mesh(),
        scratch_types=[
            pltpu.VMEM_SHARED((NP, H), jnp.float32),
            pltpu.VMEM((NB, B), jnp.int32),
            pltpu.VMEM((B, H), jnp.float32),
        ],
    )


def _hist_kernel(dst_w, ones16, zeros16):
    return _hist_kernel_fn()(dst_w, ones16, zeros16)


NBH = NB // 2


def _agg_body(hs_hbm, src_hbm, dst_hbm, zeros_hbm, out_hbm,
              acc_sh, src_v, dst_v, gbuf0, gbuf1, sem0, sem1):
    c = lax.axis_index("c")
    s = lax.axis_index("s")
    wid = c * NS + s
    pltpu.sync_copy(zeros_hbm, acc_sh.at[pl.ds(s * RPS, RPS)])
    plsc.subcore_barrier()

    @pl.loop(0, 2)
    def _(p):
        pltpu.sync_copy(src_hbm.at[wid, pl.ds(p * NBH, NBH)], src_v)
        pltpu.sync_copy(dst_hbm.at[wid, pl.ds(p * NBH, NBH)], dst_v)
        pltpu.async_copy(hs_hbm.at[src_v.at[0]], gbuf0, sem0).wait()

        @pl.loop(0, NBH - 2, step=2)
        def _(j):
            cp1 = pltpu.async_copy(hs_hbm.at[src_v.at[j + 1]], gbuf1, sem1)
            pltpu.sync_copy(gbuf0, acc_sh.at[dst_v.at[j]], add=True)
            cp1.wait()
            cp0 = pltpu.async_copy(hs_hbm.at[src_v.at[j + 2]], gbuf0, sem0)
            pltpu.sync_copy(gbuf1, acc_sh.at[dst_v.at[j + 1]], add=True)
            cp0.wait()

        cp1 = pltpu.async_copy(hs_hbm.at[src_v.at[NBH - 1]], gbuf1, sem1)
        pltpu.sync_copy(gbuf0, acc_sh.at[dst_v.at[NBH - 2]], add=True)
        cp1.wait()
        pltpu.sync_copy(gbuf1, acc_sh.at[dst_v.at[NBH - 1]], add=True)

    plsc.subcore_barrier()
    pltpu.sync_copy(
        acc_sh.at[pl.ds(s * RPS, RPS)], out_hbm.at[c, pl.ds(s * RPS, RPS)]
    )


@functools.lru_cache(maxsize=None)
def _agg_kernel_fn():
    return pl.kernel(
        _agg_body,
        out_type=jax.ShapeDtypeStruct((NC, NP, H), jnp.float32),
        mesh=_vector_mesh(),
        scratch_types=[
            pltpu.VMEM_SHARED((NP, H), jnp.float32),
            pltpu.VMEM((NBH, B), jnp.int32),
            pltpu.VMEM((NBH, B), jnp.int32),
            pltpu.VMEM((B, H), jnp.float32),
            pltpu.VMEM((B, H), jnp.float32),
            pltpu.SemaphoreType.DMA,
            pltpu.SemaphoreType.DMA,
        ],
    )


def _agg_kernel(hs, src_w, dst_w, zerosH):
    return _agg_kernel_fn()(hs, src_w, dst_w, zerosH)


def _stage1_body(x_ref, w_ref, hist_ref, dis_ref, hs_ref):
    hist = hist_ref[...]
    deg = hist[0, :, 0] + hist[1, :, 0] + 1.0
    dis = lax.rsqrt(deg)[:, None]
    dis_ref[...] = dis
    h = jnp.dot(x_ref[...], w_ref[...], preferred_element_type=jnp.float32)
    hs_ref[...] = h * dis


def _stage1(x_p, w1, hist):
    return pl.pallas_call(
        _stage1_body,
        out_shape=(
            jax.ShapeDtypeStruct((NP, 1), jnp.float32),
            jax.ShapeDtypeStruct((NP, H), jnp.float32),
        ),
        grid=(NP // BM,),
        in_specs=[
            pl.BlockSpec((BM, D), lambda i: (i, 0)),
            pl.BlockSpec((D, H), lambda i: (0, 0)),
            pl.BlockSpec((NC, BM, H), lambda i: (0, i, 0)),
        ],
        out_specs=(
            pl.BlockSpec((BM, 1), lambda i: (i, 0)),
            pl.BlockSpec((BM, H), lambda i: (i, 0)),
        ),
    )(x_p, w1, hist)


def _mid_body(p_ref, hs_ref, dis_ref, b_ref, w_ref, out_ref):
    p = p_ref[...]
    dis = dis_ref[...]
    tot = (p[0] + p[1] + hs_ref[...]) * dis + b_ref[...]
    h = jnp.maximum(tot, 0.0)
    rows = pl.program_id(0) * BM + lax.broadcasted_iota(jnp.int32, (BM, 1), 0)
    h = jnp.where(rows < N, h, 0.0)
    out_ref[...] = (
        jnp.dot(h, w_ref[...], preferred_element_type=jnp.float32) * dis
    )


def _mid_stage(partials, hs, dis, b_row, w_next):
    return pl.pallas_call(
        _mid_body,
        out_shape=jax.ShapeDtypeStruct((NP, H), jnp.float32),
        grid=(NP // BM,),
        in_specs=[
            pl.BlockSpec((NC, BM, H), lambda i: (0, i, 0)),
            pl.BlockSpec((BM, H), lambda i: (i, 0)),
            pl.BlockSpec((BM, 1), lambda i: (i, 0)),
            pl.BlockSpec((1, H), lambda i: (0, 0)),
            pl.BlockSpec((H, H), lambda i: (0, 0)),
        ],
        out_specs=pl.BlockSpec((BM, H), lambda i: (i, 0)),
    )(partials, hs, dis, b_row, w_next)


def _final_body(p_ref, hs_ref, dis_ref, b_ref, batch_ref, wl_ref, bl_ref,
                out_ref, pooled_ref):
    i = pl.program_id(0)

    @pl.when(i == 0)
    def _():
        pooled_ref[...] = jnp.zeros_like(pooled_ref)

    p = p_ref[...]
    tot = (p[0] + p[1] + hs_ref[...]) * dis_ref[...] + b_ref[...]
    h = jnp.maximum(tot, 0.0)
    bids = batch_ref[...]
    onehot = (bids == lax.broadcasted_iota(jnp.int32, (1, G), 1)).astype(
        jnp.float32
    )
    pooled_ref[...] += lax.dot_general(
        onehot, h, (((0,), (0,)), ((), ())),
        preferred_element_type=jnp.float32,
    )

    @pl.when(i == pl.num_programs(0) - 1)
    def _():
        out_ref[...] = (
            jnp.dot(pooled_ref[...], wl_ref[...],
                    preferred_element_type=jnp.float32)
            + bl_ref[...]
        )


def _final_stage(partials, hs, dis, b_row, batch_p, wl_pad, bl_pad):
    return pl.pallas_call(
        _final_body,
        out_shape=jax.ShapeDtypeStruct((G, 8), jnp.float32),
        grid=(NP // BM,),
        in_specs=[
            pl.BlockSpec((NC, BM, H), lambda i: (0, i, 0)),
            pl.BlockSpec((BM, H), lambda i: (i, 0)),
            pl.BlockSpec((BM, 1), lambda i: (i, 0)),
            pl.BlockSpec((1, H), lambda i: (0, 0)),
            pl.BlockSpec((BM, 1), lambda i: (i, 0)),
            pl.BlockSpec((H, 8), lambda i: (0, 0)),
            pl.BlockSpec((1, 8), lambda i: (0, 0)),
        ],
        out_specs=pl.BlockSpec((G, 8), lambda i: (0, 0)),
        scratch_shapes=[pltpu.VMEM((G, H), jnp.float32)],
    )(partials, hs, dis, b_row, batch_p, wl_pad, bl_pad)


@jax.jit
def kernel(x, edge_index, batch, W1, b1, W2, b2, W3, b3, Wl, bl):
    f32 = jnp.float32
    x_p = jnp.zeros((NP, D), f32).at[:N].set(x.astype(f32))
    src = edge_index[0].astype(jnp.int32)
    dst = edge_index[1].astype(jnp.int32)
    pad = jnp.full((EP - E,), N, jnp.int32)
    src_w = jnp.concatenate([src, pad]).reshape(NW, NB, B)
    dst_w = jnp.concatenate([dst, pad]).reshape(NW, NB, B)
    batch_p = jnp.concatenate(
        [batch.astype(jnp.int32), jnp.full((NP - N,), G, jnp.int32)]
    ).reshape(NP, 1)
    onesH = jnp.ones((B, H), f32)
    zerosH = jnp.zeros((RPS, H), f32)
    b1r = b1.astype(f32).reshape(1, H)
    b2r = b2.astype(f32).reshape(1, H)
    b3r = b3.astype(f32).reshape(1, H)
    wl_pad = jnp.zeros((H, 8), f32).at[:, :2].set(Wl.astype(f32))
    bl_pad = jnp.zeros((1, 8), f32).at[0, :2].set(bl.astype(f32))

    hist = _hist_kernel(dst_w, onesH, zerosH)

    dis, hs = _stage1(x_p, W1.astype(f32), hist)
    partials = _agg_kernel(hs, src_w, dst_w, zerosH)
    hs = _mid_stage(partials, hs, dis, b1r, W2.astype(f32))
    partials = _agg_kernel(hs, src_w, dst_w, zerosH)
    hs = _mid_stage(partials, hs, dis, b2r, W3.astype(f32))
    partials = _agg_kernel(hs, src_w, dst_w, zerosH)
    out = _final_stage(partials, hs, dis, b3r, batch_p, wl_pad, bl_pad)
    return out[:, :2]
